# baseline (speedup 1.0000x reference)
import functools

import jax
import jax.numpy as jnp
from jax import lax
from jax.experimental import pallas as pl
from jax.experimental.pallas import tpu as pltpu
from jax.experimental.pallas import tpu_sc as plsc

B = 1024
NCOMP = 200
D = 128
DY = 16
NPAD = 208
SIGMA = 8.0
EPS = 1e-12

_NC = 2
_NS = 16
_NW = _NC * _NS
_QPW = B // _NW

_CHUNKS = ((0, 104), (104, 96))


def _sc_gather_body(sx_hbm, sy_hbm, nbr_hbm, gx_hbm, gy_hbm,
                    idx_v, rx_v, ry_v, sem):
    wid = lax.axis_index("s") * _NC + lax.axis_index("c")
    base = wid * _QPW

    def step(i, carry):
        q = base + i
        pltpu.sync_copy(nbr_hbm.at[q], idx_v)
        cps = []
        for (o, ln) in _CHUNKS:
            cps.append(pltpu.make_async_copy(
                sx_hbm.at[idx_v.at[pl.ds(o, ln)]], rx_v.at[pl.ds(o, ln)], sem))
            cps.append(pltpu.make_async_copy(
                sy_hbm.at[idx_v.at[pl.ds(o, ln)]], ry_v.at[pl.ds(o, ln)], sem))
        for c in cps:
            c.start()
        for c in cps:
            c.wait()
        pltpu.sync_copy(rx_v, gx_hbm.at[q])
        pltpu.sync_copy(ry_v, gy_hbm.at[q])
        return carry

    lax.fori_loop(0, _QPW, step, 0)


@functools.cache
def _sc_gather():
    return pl.kernel(
        _sc_gather_body,
        mesh=plsc.VectorSubcoreMesh(
            core_axis_name="c", subcore_axis_name="s",
            num_cores=_NC, num_subcores=_NS),
        out_type=[
            jax.ShapeDtypeStruct((B, NCOMP, D), jnp.float32),
            jax.ShapeDtypeStruct((B, NCOMP, DY), jnp.float32),
        ],
        scratch_types=[
            pltpu.VMEM((NPAD,), jnp.int32),
            pltpu.VMEM((NCOMP, D), jnp.float32),
            pltpu.VMEM((NCOMP, DY), jnp.float32),
            pltpu.SemaphoreType.DMA,
        ],
        compiler_params=pltpu.CompilerParams(use_tc_tiling_on_sc=False),
    )


_QBLK = 8


def _tc_body(x_ref, gx_ref, gy_ref, out_ref):
    minor_contract = (((1,), (1,)), ((), ()))
    ones_d = jnp.ones((1, D), jnp.float32)
    ones_y = jnp.ones((1, DY), jnp.float32)
    for q in range(_QBLK):
        xq = x_ref[q][None, :]
        rows = gx_ref[q]
        dot = lax.dot_general(xq, rows, minor_contract,
                              preferred_element_type=jnp.float32)
        s2 = lax.dot_general(ones_d, rows * rows, minor_contract,
                             preferred_element_type=jnp.float32)
        x2 = jnp.sum(xq * xq)
        d2 = (x2 + s2) - 2.0 * dot
        k2 = jnp.exp(d2 * (-1.0 / (SIGMA * SIGMA)))
        y = gy_ref[q]
        y2 = y * y
        n2 = lax.dot_general(ones_y, y2, minor_contract,
                             preferred_element_type=jnp.float32)
        denom = jnp.sqrt(n2) + EPS
        coef = k2 / ((jnp.sum(k2) + EPS) * denom * denom)
        out_ref[q, :] = lax.dot_general(
            coef, y2, (((1,), (0,)), ((), ())),
            preferred_element_type=jnp.float32)[0]


def _tc_compute(x_enc, gx, gy):
    return pl.pallas_call(
        _tc_body,
        grid=(B // _QBLK,),
        in_specs=[
            pl.BlockSpec((_QBLK, D), lambda i: (i, 0)),
            pl.BlockSpec((_QBLK, NCOMP, D), lambda i: (i, 0, 0)),
            pl.BlockSpec((_QBLK, NCOMP, DY), lambda i: (i, 0, 0)),
        ],
        out_specs=pl.BlockSpec((_QBLK, DY), lambda i: (i, 0)),
        out_shape=jax.ShapeDtypeStruct((B, DY), jnp.float32),
    )(x_enc, gx, gy)


def kernel(x_enc, neighbors, samples_x, samples_y):
    sy_pad = jnp.pad(samples_y, ((0, 0), (0, DY - samples_y.shape[1])))
    nbr_pad = jnp.pad(neighbors, ((0, 0), (0, NPAD - NCOMP)))
    gx, gy = _sc_gather()(samples_x, sy_pad, nbr_pad)
    out = _tc_compute(x_enc, gx, gy)
    return out[:, :samples_y.shape[1]]


# speedup vs baseline: 1.0160x; 1.0160x over previous
import functools

import jax
import jax.numpy as jnp
from jax import lax
from jax.experimental import pallas as pl
from jax.experimental.pallas import tpu as pltpu
from jax.experimental.pallas import tpu_sc as plsc

B = 1024
NCOMP = 200
NPAD = 208
D = 128
DY = 10
DYP = 16
DOUT = 16
SIGMA = 8.0
EPS = 1e-12

_NC = 2
_NS = 16
_NW = _NC * _NS
_QPW = B // _NW
_NG = NPAD // 16

L16 = 16


def _rsqrt_nr(s):
    i = lax.bitcast_convert_type(s, jnp.int32)
    i = 0x5F3759DF - lax.shift_right_arithmetic(i, jnp.full((L16,), 1, jnp.int32))
    r = lax.bitcast_convert_type(i, jnp.float32)
    for _ in range(3):
        r = r * (1.5 - 0.5 * s * r * r)
    return r


def _sc_body(x_hbm, nbr_hbm, sx_hbm, sy_hbm, out_hbm,
             idx_v, x_v, rx_v, ry_v, d2_v, k2_v, stage_v, sem):
    wid = lax.axis_index("s") * _NC + lax.axis_index("c")
    base = wid * _QPW
    iota = lax.iota(jnp.int32, L16)
    zeros = jnp.zeros((L16,), jnp.float32)

    def q_step(qi, carry):
        q = base + qi
        cx = pltpu.make_async_copy(x_hbm.at[pl.ds(q, 1)], x_v, sem)
        cx.start()
        pltpu.sync_copy(nbr_hbm.at[q], idx_v)
        cps = []
        for (o, ln) in ((0, 104), (104, 104)):
            cps.append(pltpu.make_async_copy(
                sx_hbm.at[idx_v.at[pl.ds(o, ln)]], rx_v.at[pl.ds(o, ln)], sem))
            cps.append(pltpu.make_async_copy(
                sy_hbm.at[idx_v.at[pl.ds(o, ln)]], ry_v.at[pl.ds(o, ln)], sem))
        for c in cps:
            c.start()
        cx.wait()
        for c in cps:
            c.wait()

        for g in range(_NG):
            d2_v[pl.ds(g * L16, L16)] = zeros

        def dc_step(dc, carry2):
            dbase = dc * L16
            zidx = jnp.zeros((L16,), jnp.int32)
            xvs = [plsc.load_gather(
                x_v, [zidx, jnp.full((L16,), dbase + k, jnp.int32)])
                for k in range(L16)]
            for g in range(_NG):
                nvec = iota + g * L16
                accs = [d2_v[pl.ds(g * L16, L16)], zeros, zeros, zeros]
                for k in range(L16):
                    dvec = jnp.full((L16,), dbase + k, jnp.int32)
                    sv = plsc.load_gather(rx_v, [nvec, dvec])
                    diff = sv - xvs[k]
                    accs[k % 4] = accs[k % 4] + diff * diff
                d2_v[pl.ds(g * L16, L16)] = (
                    (accs[0] + accs[1]) + (accs[2] + accs[3]))
            return carry2

        lax.fori_loop(0, D // L16, dc_step, 0)

        tot = zeros
        for g in range(_NG):
            k2 = jnp.exp(d2_v[pl.ds(g * L16, L16)] * (-1.0 / (SIGMA * SIGMA)))
            if g == _NG - 1:
                k2 = jnp.where(iota < L16 - (NPAD - NCOMP), k2, 0.0)
            k2_v[pl.ds(g * L16, L16)] = k2
            tot = tot + k2
        s = jnp.sum(tot)
        invt = 1.0 / (lax.broadcast_in_dim(s, (L16,), ()) + EPS)

        acc_y = [zeros] * DY
        for g in range(_NG):
            nvec = iota + g * L16
            k2g = k2_v[pl.ds(g * L16, L16)]
            c2 = []
            sy0, sy1 = zeros, zeros
            for d in range(DY):
                col = plsc.load_gather(
                    ry_v, [nvec, jnp.full((L16,), d, jnp.int32)])
                c2d = col * col
                c2.append(c2d)
                if d % 2 == 0:
                    sy0 = sy0 + c2d
                else:
                    sy1 = sy1 + c2d
            n2 = sy0 + sy1
            norm = n2 * _rsqrt_nr(n2)
            denom = norm + EPS
            coef = k2g * invt / (denom * denom)
            for d in range(DY):
                acc_y[d] = acc_y[d] + coef * c2[d]

        outv = zeros
        for d in range(DY):
            sd = jnp.sum(acc_y[d])
            outv = outv + jnp.where(
                iota == d, lax.broadcast_in_dim(sd, (L16,), ()), 0.0)
        stage_v[...] = outv
        pltpu.sync_copy(stage_v, out_hbm.at[q])
        return carry

    lax.fori_loop(0, _QPW, q_step, 0)


@functools.cache
def _sc_kernel():
    return pl.kernel(
        _sc_body,
        mesh=plsc.VectorSubcoreMesh(
            core_axis_name="c", subcore_axis_name="s",
            num_cores=_NC, num_subcores=_NS),
        out_type=jax.ShapeDtypeStruct((B, DOUT), jnp.float32),
        scratch_types=[
            pltpu.VMEM((NPAD,), jnp.int32),
            pltpu.VMEM((1, D), jnp.float32),
            pltpu.VMEM((NPAD, D), jnp.float32),
            pltpu.VMEM((NPAD, DYP), jnp.float32),
            pltpu.VMEM((NPAD,), jnp.float32),
            pltpu.VMEM((NPAD,), jnp.float32),
            pltpu.VMEM((L16,), jnp.float32),
            pltpu.SemaphoreType.DMA,
        ],
        compiler_params=pltpu.CompilerParams(
            use_tc_tiling_on_sc=False, needs_layout_passes=False),
    )


def kernel(x_enc, neighbors, samples_x, samples_y):
    nbr_pad = jnp.pad(neighbors, ((0, 0), (0, NPAD - NCOMP)))
    sy_pad = jnp.pad(samples_y, ((0, 0), (0, DYP - DY)))
    out = _sc_kernel()(x_enc, nbr_pad, samples_x, sy_pad)
    return out[:, :DY]


# speedup vs baseline: 1.3002x; 1.2798x over previous
import functools

import jax
import jax.numpy as jnp
from jax import lax
from jax.experimental import pallas as pl
from jax.experimental.pallas import tpu as pltpu
from jax.experimental.pallas import tpu_sc as plsc

B = 1024
NCOMP = 200
NPAD = 208
D = 128
DY = 10
DYP = 16
DOUT = 16
SIGMA = 8.0
EPS = 1e-12

_NC = 2
_NS = 16
_NW = _NC * _NS
_QPW = B // _NW
_NG = NPAD // 16

L16 = 16


def _rsqrt_nr(s):
    i = lax.bitcast_convert_type(s, jnp.int32)
    i = 0x5F3759DF - lax.shift_right_arithmetic(i, jnp.full((L16,), 1, jnp.int32))
    r = lax.bitcast_convert_type(i, jnp.float32)
    for _ in range(3):
        r = r * (1.5 - 0.5 * s * r * r)
    return r


def _sc_body(x_hbm, nbr_hbm, sx_hbm, sy_hbm, out_hbm,
             idx_v, x_v, rx_v, ry_v, k2_v, stage_v, sem):
    wid = lax.axis_index("s") * _NC + lax.axis_index("c")
    base = wid * _QPW
    iota = lax.iota(jnp.int32, L16)
    zeros = jnp.zeros((L16,), jnp.float32)

    def q_step(qi, carry):
        q = base + qi
        cx = pltpu.make_async_copy(x_hbm.at[pl.ds(q, 1)], x_v, sem)
        cx.start()
        pltpu.sync_copy(nbr_hbm.at[q], idx_v)
        cps = []
        for (o, ln) in ((0, 104), (104, 104)):
            cps.append(pltpu.make_async_copy(
                sx_hbm.at[idx_v.at[pl.ds(o, ln)]], rx_v.at[pl.ds(o, ln)], sem))
            cps.append(pltpu.make_async_copy(
                sy_hbm.at[idx_v.at[pl.ds(o, ln)]], ry_v.at[pl.ds(o, ln)], sem))
        for c in cps:
            c.start()
        cx.wait()
        for c in cps:
            c.wait()

        xc = [x_v[0, pl.ds(c * L16, L16)] for c in range(D // L16)]

        def ab_step(g, tot_c):
            d2g = zeros
            for l in range(L16):
                n = g * L16 + l
                p0, p1 = zeros, zeros
                for c in range(D // L16):
                    rv = rx_v[n, pl.ds(c * L16, L16)]
                    diff = rv - xc[c]
                    if c % 2 == 0:
                        p0 = p0 + diff * diff
                    else:
                        p1 = p1 + diff * diff
                sd2 = jnp.sum(p0 + p1)
                d2g = d2g + jnp.where(
                    iota == l, lax.broadcast_in_dim(sd2, (L16,), ()), 0.0)
            nvec = iota + g * L16
            k2 = jnp.exp(d2g * (-1.0 / (SIGMA * SIGMA)))
            k2 = jnp.where(nvec < NCOMP, k2, 0.0)
            k2_v[pl.ds(g * L16, L16)] = k2
            return tot_c + k2

        tot = lax.fori_loop(0, _NG, ab_step, zeros)
        s = jnp.sum(tot)
        invt = 1.0 / (lax.broadcast_in_dim(s, (L16,), ()) + EPS)

        def c_step(g, accs):
            k2g = k2_v[pl.ds(g * L16, L16)]
            n2 = zeros
            for l in range(L16):
                yrow = ry_v[g * L16 + l, pl.ds(0, L16)]
                sl = jnp.sum(yrow * yrow)
                n2 = n2 + jnp.where(
                    iota == l, lax.broadcast_in_dim(sl, (L16,), ()), 0.0)
            norm = n2 * _rsqrt_nr(n2)
            denom = norm + EPS
            coef = k2g * invt / (denom * denom)
            out_g = accs
            for l in range(L16):
                cl = jnp.sum(jnp.where(iota == l, coef, 0.0))
                yrow = ry_v[g * L16 + l, pl.ds(0, L16)]
                out_g = out_g + (
                    lax.broadcast_in_dim(cl, (L16,), ()) * yrow * yrow)
            return out_g

        outv = lax.fori_loop(0, _NG, c_step, zeros)
        stage_v[...] = outv
        pltpu.sync_copy(stage_v, out_hbm.at[q])
        return carry

    lax.fori_loop(0, _QPW, q_step, 0)


@functools.cache
def _sc_kernel():
    return pl.kernel(
        _sc_body,
        mesh=plsc.VectorSubcoreMesh(
            core_axis_name="c", subcore_axis_name="s",
            num_cores=_NC, num_subcores=_NS),
        out_type=jax.ShapeDtypeStruct((B, DOUT), jnp.float32),
        scratch_types=[
            pltpu.VMEM((NPAD,), jnp.int32),
            pltpu.VMEM((1, D), jnp.float32),
            pltpu.VMEM((NPAD, D), jnp.float32),
            pltpu.VMEM((NPAD, DYP), jnp.float32),
            pltpu.VMEM((NPAD,), jnp.float32),
            pltpu.VMEM((L16,), jnp.float32),
            pltpu.SemaphoreType.DMA,
        ],
        compiler_params=pltpu.CompilerParams(
            use_tc_tiling_on_sc=False, needs_layout_passes=False),
    )


def kernel(x_enc, neighbors, samples_x, samples_y):
    nbr_pad = jnp.pad(neighbors, ((0, 0), (0, NPAD - NCOMP)))
    sy_pad = jnp.pad(samples_y, ((0, 0), (0, DYP - DY)))
    out = _sc_kernel()(x_enc, nbr_pad, samples_x, sy_pad)
    return out[:, :DY]


# speedup vs baseline: 1.3026x; 1.0018x over previous
import functools

import jax
import jax.numpy as jnp
from jax import lax
from jax.experimental import pallas as pl
from jax.experimental.pallas import tpu as pltpu
from jax.experimental.pallas import tpu_sc as plsc

B = 1024
NCOMP = 200
NPAD = 208
D = 128
DY = 10
DYP = 16
DOUT = 16
SIGMA = 8.0
EPS = 1e-12

_NC = 2
_NS = 16
_NW = _NC * _NS
_QPW = B // _NW
_NG = NPAD // 16

L16 = 16


def _rsqrt_nr(s):
    i = lax.bitcast_convert_type(s, jnp.int32)
    i = 0x5F3759DF - lax.shift_right_arithmetic(i, jnp.full((L16,), 1, jnp.int32))
    r = lax.bitcast_convert_type(i, jnp.float32)
    for _ in range(3):
        r = r * (1.5 - 0.5 * s * r * r)
    return r


def _sc_body(x_hbm, nbr_hbm, sx_hbm, sy_hbm, out_hbm,
             idx_v, x_v, rx_v, ry_v, k2_v, coef_v, stage_v, sem):
    wid = lax.axis_index("s") * _NC + lax.axis_index("c")
    base = wid * _QPW
    iota = lax.iota(jnp.int32, L16)
    zeros = jnp.zeros((L16,), jnp.float32)

    def q_step(qi, carry):
        q = base + qi
        cx = pltpu.make_async_copy(x_hbm.at[pl.ds(q, 1)], x_v, sem)
        cx.start()
        pltpu.sync_copy(nbr_hbm.at[q], idx_v)
        cps = []
        for (o, ln) in ((0, 104), (104, 104)):
            cps.append(pltpu.make_async_copy(
                sx_hbm.at[idx_v.at[pl.ds(o, ln)]], rx_v.at[pl.ds(o, ln)], sem))
            cps.append(pltpu.make_async_copy(
                sy_hbm.at[idx_v.at[pl.ds(o, ln)]], ry_v.at[pl.ds(o, ln)], sem))
        for c in cps:
            c.start()
        cx.wait()
        for c in cps:
            c.wait()

        xc = [x_v[0, pl.ds(c * L16, L16)] for c in range(D // L16)]

        def ab_step(g, tot_c):
            d2p = [zeros, zeros, zeros, zeros]
            for l in range(L16):
                n = g * L16 + l
                p = [zeros, zeros, zeros, zeros]
                for c in range(D // L16):
                    rv = rx_v[n, pl.ds(c * L16, L16)]
                    diff = rv - xc[c]
                    p[c % 4] = p[c % 4] + diff * diff
                sd2 = jnp.sum((p[0] + p[1]) + (p[2] + p[3]))
                d2p[l % 4] = d2p[l % 4] + jnp.where(
                    iota == l, lax.broadcast_in_dim(sd2, (L16,), ()), 0.0)
            d2g = (d2p[0] + d2p[1]) + (d2p[2] + d2p[3])
            nvec = iota + g * L16
            k2 = jnp.exp(d2g * (-1.0 / (SIGMA * SIGMA)))
            k2 = jnp.where(nvec < NCOMP, k2, 0.0)
            k2_v[pl.ds(g * L16, L16)] = k2
            return tot_c + k2

        tot = lax.fori_loop(0, _NG, ab_step, zeros)
        s = jnp.sum(tot)
        invt = 1.0 / (lax.broadcast_in_dim(s, (L16,), ()) + EPS)

        def c_step(g, accs):
            k2g = k2_v[pl.ds(g * L16, L16)]
            n2p = [zeros, zeros, zeros, zeros]
            for l in range(L16):
                yrow = ry_v[g * L16 + l, pl.ds(0, L16)]
                sl = jnp.sum(yrow * yrow)
                n2p[l % 4] = n2p[l % 4] + jnp.where(
                    iota == l, lax.broadcast_in_dim(sl, (L16,), ()), 0.0)
            n2 = (n2p[0] + n2p[1]) + (n2p[2] + n2p[3])
            norm = n2 * _rsqrt_nr(n2)
            denom = norm + EPS
            coef = k2g * invt / (denom * denom)
            parts = list(accs)
            for l in range(L16):
                cl = jnp.sum(jnp.where(iota == l, coef, 0.0))
                yrow = ry_v[g * L16 + l, pl.ds(0, L16)]
                parts[l % 4] = parts[l % 4] + (
                    lax.broadcast_in_dim(cl, (L16,), ()) * yrow * yrow)
            return tuple(parts)

        outp = lax.fori_loop(0, _NG, c_step, (zeros, zeros, zeros, zeros))
        outv = (outp[0] + outp[1]) + (outp[2] + outp[3])
        stage_v[...] = outv
        pltpu.sync_copy(stage_v, out_hbm.at[q])
        return carry

    lax.fori_loop(0, _QPW, q_step, 0)


@functools.cache
def _sc_kernel():
    return pl.kernel(
        _sc_body,
        mesh=plsc.VectorSubcoreMesh(
            core_axis_name="c", subcore_axis_name="s",
            num_cores=_NC, num_subcores=_NS),
        out_type=jax.ShapeDtypeStruct((B, DOUT), jnp.float32),
        scratch_types=[
            pltpu.VMEM((NPAD,), jnp.int32),
            pltpu.VMEM((1, D), jnp.float32),
            pltpu.VMEM((NPAD, D), jnp.float32),
            pltpu.VMEM((NPAD, DYP), jnp.float32),
            pltpu.VMEM((NPAD,), jnp.float32),
            pltpu.VMEM((L16,), jnp.float32),
            pltpu.VMEM((L16,), jnp.float32),
            pltpu.SemaphoreType.DMA,
        ],
        compiler_params=pltpu.CompilerParams(
            use_tc_tiling_on_sc=False, needs_layout_passes=False),
    )


def kernel(x_enc, neighbors, samples_x, samples_y):
    nbr_pad = jnp.pad(neighbors, ((0, 0), (0, NPAD - NCOMP)))
    sy_pad = jnp.pad(samples_y, ((0, 0), (0, DYP - DY)))
    out = _sc_kernel()(x_enc, nbr_pad, samples_x, sy_pad)
    return out[:, :DY]
